# Initial kernel scaffold; baseline (speedup 1.0000x reference)
import jax
import jax.numpy as jnp
from jax.experimental import pallas as pl


def kernel(x, Wq, gamma, beta, keyl, keyr, value_table):
    raise NotImplementedError("write your pallas kernel here")


# speedup vs baseline: 3.7877x; 3.7877x over previous
import functools

import jax
import jax.numpy as jnp
from jax import lax
from jax.experimental import pallas as pl
from jax.experimental.pallas import tpu as pltpu
from jax.experimental.pallas import tpu_sc as plsc

DIM_IN = 1024
DIM_HIDDEN = 512
NUM_SUBKEYS = 256
TOP_K = 16
NUM_HEADS = 4
SUBKEY = DIM_HIDDEN // 2
CTX = 2048
EPS = 1e-5
NFEAT = DIM_HIDDEN * NUM_HEADS

TOK_BLK = 256
NUM_BLKS = CTX // TOK_BLK

NEG = -3.0e38


def _qstats_kernel(x_ref, wq_ref, q_ref, stats_ref):
    step = pl.program_id(0)
    qb = lax.dot_general(x_ref[...], wq_ref[...], (((1,), (1,)), ((), ())),
                         preferred_element_type=jnp.float32)
    q_ref[...] = qb

    @pl.when(step == 0)
    def _():
        stats_ref[...] = jnp.zeros_like(stats_ref)

    s1 = jnp.sum(qb, axis=0, keepdims=True)
    s2 = jnp.sum(qb * qb, axis=0, keepdims=True)
    stats_ref[...] += jnp.concatenate([s1, s2], axis=0)


def _top16(scores, colids):
    vals, idxs = [], []
    sc = scores
    for _ in range(TOP_K):
        m = jnp.max(sc, axis=1, keepdims=True)
        am = jnp.min(jnp.where(sc >= m, colids, jnp.int32(1 << 30)),
                     axis=1, keepdims=True)
        vals.append(m)
        idxs.append(am)
        sc = jnp.where(colids == am, NEG, sc)
    return jnp.concatenate(vals, axis=1), jnp.concatenate(idxs, axis=1)


def _select_kernel(q_ref, stats_ref, gamma_ref, beta_ref, kl_ref, kr_ref,
                   vidx_ref, wsp_ref):
    n = jnp.float32(CTX)
    mean = stats_ref[0:1, :] / n
    var = stats_ref[1:2, :] / n - mean * mean
    scale = gamma_ref[0:1, :] * lax.rsqrt(var + EPS)
    qn = (q_ref[...] - mean) * scale + beta_ref[0:1, :]

    colk = lax.broadcasted_iota(jnp.int32, (TOK_BLK, NUM_SUBKEYS), 1)
    colp = lax.broadcasted_iota(jnp.int32, (TOK_BLK, TOP_K * TOP_K), 1)
    prow = lax.broadcasted_iota(jnp.int32, (TOP_K, TOP_K * TOP_K), 1)
    krow = lax.broadcasted_iota(jnp.int32, (TOP_K, TOP_K * TOP_K), 0)
    el = (prow // TOP_K == krow).astype(jnp.float32)
    er = (prow % TOP_K == krow).astype(jnp.float32)
    iot16 = lax.broadcasted_iota(jnp.int32, (TOK_BLK, TOP_K), 1)

    for h in range(NUM_HEADS):
        ql = qn[:, h * DIM_HIDDEN:h * DIM_HIDDEN + SUBKEY]
        qr = qn[:, h * DIM_HIDDEN + SUBKEY:(h + 1) * DIM_HIDDEN]
        scl = lax.dot_general(ql, kl_ref[h], (((1,), (1,)), ((), ())),
                              preferred_element_type=jnp.float32)
        scr = lax.dot_general(qr, kr_ref[h], (((1,), (1,)), ((), ())),
                              preferred_element_type=jnp.float32)
        vl, il = _top16(scl, colk)
        vr, ir = _top16(scr, colk)

        ps = (lax.dot_general(vl, el, (((1,), (0,)), ((), ())),
                              preferred_element_type=jnp.float32,
                              precision=lax.Precision.HIGHEST) +
              lax.dot_general(vr, er, (((1,), (0,)), ((), ())),
                              preferred_element_type=jnp.float32,
                              precision=lax.Precision.HIGHEST))
        pv, pp = _top16(ps, colp)

        pm = jnp.max(pv, axis=1, keepdims=True)
        e = jnp.exp(pv - pm)
        w = e / jnp.sum(e, axis=1, keepdims=True)

        pi = pp // TOP_K
        wagg = jnp.zeros((TOK_BLK, TOP_K), jnp.float32)
        for k in range(TOP_K):
            wagg = wagg + jnp.where(pi[:, k:k + 1] == iot16,
                                    w[:, k:k + 1], 0.0)

        vidx_ref[:, h * TOP_K:(h + 1) * TOP_K] = il * NUM_SUBKEYS + ir
        wsp_ref[:, (h * TOP_K) * 16:((h + 1) * TOP_K) * 16] = (
            jnp.repeat(wagg, 16, axis=1))


TOK_PER_W = 64


def _sc_gather_body(vt_hbm, idx_hbm, w_hbm, out_hbm,
                    idx_v, buf_v, w_v, acc_v, gsem, wsem):
    nc = 2
    wid = lax.axis_index("s") * nc + lax.axis_index("c")
    base = wid * TOK_PER_W
    pltpu.sync_copy(idx_hbm.at[pl.ds(base, TOK_PER_W)], idx_v)

    def token_body(t, carry):
        pltpu.async_copy(vt_hbm.at[idx_v.at[t]], buf_v, gsem)
        pltpu.async_copy(w_hbm.at[base + t], w_v, wsem)
        pltpu.make_async_copy(vt_hbm.at[idx_v.at[t]], buf_v, gsem).wait()
        pltpu.make_async_copy(w_hbm.at[base + t], w_v, wsem).wait()
        zero = jnp.zeros((16,), jnp.float32)
        for cg in range(4):
            def row_body(r, accs):
                wv = w_v[r]
                return tuple(
                    accs[g] + wv * buf_v[r, pl.ds((cg * 16 + g) * 16, 16)]
                    for g in range(16))
            accs = lax.fori_loop(0, 64, row_body, (zero,) * 16)
            for g in range(16):
                acc_v[pl.ds((cg * 16 + g) * 16, 16)] = accs[g]
        pltpu.sync_copy(acc_v, out_hbm.at[base + t])
        return carry

    lax.fori_loop(0, TOK_PER_W, token_body, 0)


def _sc_gather(value_table, vidx, wsp):
    mesh = plsc.VectorSubcoreMesh(core_axis_name="c", subcore_axis_name="s")
    kern = functools.partial(
        pl.kernel,
        mesh=mesh,
        out_type=jax.ShapeDtypeStruct((CTX, DIM_IN), jnp.float32),
        scratch_types=[
            pltpu.VMEM((TOK_PER_W, 64), jnp.int32),
            pltpu.VMEM((64, DIM_IN), jnp.float32),
            pltpu.VMEM((64, 16), jnp.float32),
            pltpu.VMEM((DIM_IN,), jnp.float32),
            pltpu.SemaphoreType.DMA,
            pltpu.SemaphoreType.DMA,
        ],
    )(_sc_gather_body)
    return kern(value_table, vidx, wsp)


def kernel(x, Wq, gamma, beta, keyl, keyr, value_table):
    b, c, _ = x.shape
    x2 = x.reshape(CTX, DIM_IN)

    q, stats = pl.pallas_call(
        _qstats_kernel,
        grid=(NUM_BLKS,),
        in_specs=[
            pl.BlockSpec((TOK_BLK, DIM_IN), lambda i: (i, 0)),
            pl.BlockSpec((NFEAT, DIM_IN), lambda i: (0, 0)),
        ],
        out_specs=[
            pl.BlockSpec((TOK_BLK, NFEAT), lambda i: (i, 0)),
            pl.BlockSpec((2, NFEAT), lambda i: (0, 0)),
        ],
        out_shape=[
            jax.ShapeDtypeStruct((CTX, NFEAT), jnp.float32),
            jax.ShapeDtypeStruct((2, NFEAT), jnp.float32),
        ],
    )(x2, Wq)

    vidx, wsp = pl.pallas_call(
        _select_kernel,
        grid=(NUM_BLKS,),
        in_specs=[
            pl.BlockSpec((TOK_BLK, NFEAT), lambda i: (i, 0)),
            pl.BlockSpec((2, NFEAT), lambda i: (0, 0)),
            pl.BlockSpec((1, NFEAT), lambda i: (0, 0)),
            pl.BlockSpec((1, NFEAT), lambda i: (0, 0)),
            pl.BlockSpec((NUM_HEADS, NUM_SUBKEYS, SUBKEY), lambda i: (0, 0, 0)),
            pl.BlockSpec((NUM_HEADS, NUM_SUBKEYS, SUBKEY), lambda i: (0, 0, 0)),
        ],
        out_specs=[
            pl.BlockSpec((TOK_BLK, NUM_HEADS * TOP_K), lambda i: (i, 0)),
            pl.BlockSpec((TOK_BLK, NUM_HEADS * TOP_K * 16), lambda i: (i, 0)),
        ],
        out_shape=[
            jax.ShapeDtypeStruct((CTX, NUM_HEADS * TOP_K), jnp.int32),
            jax.ShapeDtypeStruct((CTX, NUM_HEADS * TOP_K * 16), jnp.float32),
        ],
    )(q, stats, gamma.reshape(1, NFEAT), beta.reshape(1, NFEAT), keyl, keyr)

    wsp3 = wsp.reshape(CTX, NUM_HEADS * TOP_K, 16)
    out = _sc_gather(value_table, vidx, wsp3)
    return out.reshape(b, c, DIM_IN)


# speedup vs baseline: 4.4800x; 1.1828x over previous
import functools

import jax
import jax.numpy as jnp
from jax import lax
from jax.experimental import pallas as pl
from jax.experimental.pallas import tpu as pltpu
from jax.experimental.pallas import tpu_sc as plsc

DIM_IN = 1024
DIM_HIDDEN = 512
NUM_SUBKEYS = 256
TOP_K = 16
NUM_HEADS = 4
SUBKEY = DIM_HIDDEN // 2
CTX = 2048
EPS = 1e-5
NFEAT = DIM_HIDDEN * NUM_HEADS

TOK_BLK = 256
NUM_BLKS = CTX // TOK_BLK

NEG = -3.0e38


def _qstats_kernel(x_ref, wq_ref, q_ref, stats_ref):
    step = pl.program_id(0)
    qb = lax.dot_general(x_ref[...], wq_ref[...], (((1,), (1,)), ((), ())),
                         preferred_element_type=jnp.float32)
    q_ref[...] = qb

    @pl.when(step == 0)
    def _():
        stats_ref[...] = jnp.zeros_like(stats_ref)

    s1 = jnp.sum(qb, axis=0, keepdims=True)
    s2 = jnp.sum(qb * qb, axis=0, keepdims=True)
    stats_ref[...] += jnp.concatenate([s1, s2], axis=0)


def _top16(scores, colids):
    vals, idxs = [], []
    sc = scores
    for _ in range(TOP_K):
        m = jnp.max(sc, axis=1, keepdims=True)
        am = jnp.min(jnp.where(sc >= m, colids, jnp.int32(1 << 30)),
                     axis=1, keepdims=True)
        vals.append(m)
        idxs.append(am)
        sc = jnp.where(colids == am, NEG, sc)
    return jnp.concatenate(vals, axis=1), jnp.concatenate(idxs, axis=1)


def _select_kernel(q_ref, stats_ref, gamma_ref, beta_ref, kl_ref, kr_ref,
                   vidx_ref, wsp_ref):
    n = jnp.float32(CTX)
    mean = stats_ref[0:1, :] / n
    var = stats_ref[1:2, :] / n - mean * mean
    scale = gamma_ref[0:1, :] * lax.rsqrt(var + EPS)
    qn = (q_ref[...] - mean) * scale + beta_ref[0:1, :]

    colk = lax.broadcasted_iota(jnp.int32, (TOK_BLK, NUM_SUBKEYS), 1)
    colp = lax.broadcasted_iota(jnp.int32, (TOK_BLK, TOP_K * TOP_K), 1)
    prow = lax.broadcasted_iota(jnp.int32, (TOP_K, TOP_K * TOP_K), 1)
    krow = lax.broadcasted_iota(jnp.int32, (TOP_K, TOP_K * TOP_K), 0)
    el = (prow // TOP_K == krow).astype(jnp.float32)
    er = (prow % TOP_K == krow).astype(jnp.float32)
    iot16 = lax.broadcasted_iota(jnp.int32, (TOK_BLK, TOP_K), 1)

    for h in range(NUM_HEADS):
        ql = qn[:, h * DIM_HIDDEN:h * DIM_HIDDEN + SUBKEY]
        qr = qn[:, h * DIM_HIDDEN + SUBKEY:(h + 1) * DIM_HIDDEN]
        scl = lax.dot_general(ql, kl_ref[h], (((1,), (1,)), ((), ())),
                              preferred_element_type=jnp.float32)
        scr = lax.dot_general(qr, kr_ref[h], (((1,), (1,)), ((), ())),
                              preferred_element_type=jnp.float32)
        vl, il = _top16(scl, colk)
        vr, ir = _top16(scr, colk)

        ps = (lax.dot_general(vl, el, (((1,), (0,)), ((), ())),
                              preferred_element_type=jnp.float32,
                              precision=lax.Precision.HIGHEST) +
              lax.dot_general(vr, er, (((1,), (0,)), ((), ())),
                              preferred_element_type=jnp.float32,
                              precision=lax.Precision.HIGHEST))
        pv, pp = _top16(ps, colp)

        pm = jnp.max(pv, axis=1, keepdims=True)
        e = jnp.exp(pv - pm)
        w = e / jnp.sum(e, axis=1, keepdims=True)

        pi = pp // TOP_K
        wagg = jnp.zeros((TOK_BLK, TOP_K), jnp.float32)
        for k in range(TOP_K):
            wagg = wagg + jnp.where(pi[:, k:k + 1] == iot16,
                                    w[:, k:k + 1], 0.0)

        vidx_ref[:, h * TOP_K:(h + 1) * TOP_K] = il * NUM_SUBKEYS + ir
        wsp_ref[:, (h * TOP_K) * 16:((h + 1) * TOP_K) * 16] = (
            jnp.repeat(wagg, 16, axis=1))


TOK_PER_W = 64


def _sc_gather_body(vt_hbm, idx_hbm, w_hbm, out_hbm,
                    idx_v, buf0, buf1, wv0, wv1, acc_v,
                    g0, g1, ws0, ws1):
    nc = 2
    wid = lax.axis_index("s") * nc + lax.axis_index("c")
    base = wid * TOK_PER_W
    pltpu.sync_copy(idx_hbm.at[pl.ds(base, TOK_PER_W)], idx_v)
    zero = jnp.zeros((16,), jnp.float32)

    def fire_half0(t):
        pltpu.async_copy(vt_hbm.at[idx_v.at[t, pl.ds(0, 32)]], buf0, g0)

    def fire_half1(t):
        pltpu.async_copy(vt_hbm.at[idx_v.at[t, pl.ds(32, 32)]], buf1, g1)

    def accum(buf, w_v, roff, first):
        for cg in range(4):
            def row_body(r, accs):
                wv = w_v[roff + r]
                return tuple(
                    accs[g] + wv * buf[r, pl.ds((cg * 16 + g) * 16, 16)]
                    for g in range(16))
            accs = lax.fori_loop(0, 32, row_body, (zero,) * 16)
            for g in range(16):
                sl = pl.ds((cg * 16 + g) * 16, 16)
                if first:
                    acc_v[sl] = accs[g]
                else:
                    acc_v[sl] += accs[g]

    def one_token(t, w_v, wsem, wsem_next, wv_next):
        pltpu.make_async_copy(vt_hbm.at[idx_v.at[t, pl.ds(0, 32)]],
                              buf0, g0).wait()
        pltpu.make_async_copy(w_hbm.at[base + t], w_v, wsem).wait()
        fire_half1(t)
        accum(buf0, w_v, 0, True)

        @pl.when(t + 1 < TOK_PER_W)
        def _():
            fire_half0(t + 1)
            pltpu.async_copy(w_hbm.at[base + t + 1], wv_next, wsem_next)

        pltpu.make_async_copy(vt_hbm.at[idx_v.at[t, pl.ds(32, 32)]],
                              buf1, g1).wait()
        accum(buf1, w_v, 32, False)
        pltpu.sync_copy(acc_v, out_hbm.at[base + t])

    fire_half0(0)
    pltpu.async_copy(w_hbm.at[base], wv0, ws0)

    def pair_body(u, carry):
        one_token(2 * u, wv0, ws0, ws1, wv1)
        one_token(2 * u + 1, wv1, ws1, ws0, wv0)
        return carry

    lax.fori_loop(0, TOK_PER_W // 2, pair_body, 0)


def _sc_gather(value_table, vidx, wsp):
    mesh = plsc.VectorSubcoreMesh(core_axis_name="c", subcore_axis_name="s")
    kern = functools.partial(
        pl.kernel,
        mesh=mesh,
        out_type=jax.ShapeDtypeStruct((CTX, DIM_IN), jnp.float32),
        scratch_types=[
            pltpu.VMEM((TOK_PER_W, 64), jnp.int32),
            pltpu.VMEM((32, DIM_IN), jnp.float32),
            pltpu.VMEM((32, DIM_IN), jnp.float32),
            pltpu.VMEM((64, 16), jnp.float32),
            pltpu.VMEM((64, 16), jnp.float32),
            pltpu.VMEM((DIM_IN,), jnp.float32),
            pltpu.SemaphoreType.DMA,
            pltpu.SemaphoreType.DMA,
            pltpu.SemaphoreType.DMA,
            pltpu.SemaphoreType.DMA,
        ],
    )(_sc_gather_body)
    return kern(value_table, vidx, wsp)


def kernel(x, Wq, gamma, beta, keyl, keyr, value_table):
    b, c, _ = x.shape
    x2 = x.reshape(CTX, DIM_IN)

    q, stats = pl.pallas_call(
        _qstats_kernel,
        grid=(NUM_BLKS,),
        in_specs=[
            pl.BlockSpec((TOK_BLK, DIM_IN), lambda i: (i, 0)),
            pl.BlockSpec((NFEAT, DIM_IN), lambda i: (0, 0)),
        ],
        out_specs=[
            pl.BlockSpec((TOK_BLK, NFEAT), lambda i: (i, 0)),
            pl.BlockSpec((2, NFEAT), lambda i: (0, 0)),
        ],
        out_shape=[
            jax.ShapeDtypeStruct((CTX, NFEAT), jnp.float32),
            jax.ShapeDtypeStruct((2, NFEAT), jnp.float32),
        ],
    )(x2, Wq)

    vidx, wsp = pl.pallas_call(
        _select_kernel,
        grid=(NUM_BLKS,),
        in_specs=[
            pl.BlockSpec((TOK_BLK, NFEAT), lambda i: (i, 0)),
            pl.BlockSpec((2, NFEAT), lambda i: (0, 0)),
            pl.BlockSpec((1, NFEAT), lambda i: (0, 0)),
            pl.BlockSpec((1, NFEAT), lambda i: (0, 0)),
            pl.BlockSpec((NUM_HEADS, NUM_SUBKEYS, SUBKEY), lambda i: (0, 0, 0)),
            pl.BlockSpec((NUM_HEADS, NUM_SUBKEYS, SUBKEY), lambda i: (0, 0, 0)),
        ],
        out_specs=[
            pl.BlockSpec((TOK_BLK, NUM_HEADS * TOP_K), lambda i: (i, 0)),
            pl.BlockSpec((TOK_BLK, NUM_HEADS * TOP_K * 16), lambda i: (i, 0)),
        ],
        out_shape=[
            jax.ShapeDtypeStruct((CTX, NUM_HEADS * TOP_K), jnp.int32),
            jax.ShapeDtypeStruct((CTX, NUM_HEADS * TOP_K * 16), jnp.float32),
        ],
    )(q, stats, gamma.reshape(1, NFEAT), beta.reshape(1, NFEAT), keyl, keyr)

    wsp3 = wsp.reshape(CTX, NUM_HEADS * TOP_K, 16)
    out = _sc_gather(value_table, vidx, wsp3)
    return out.reshape(b, c, DIM_IN)


# speedup vs baseline: 7.8634x; 1.7552x over previous
import functools

import jax
import jax.numpy as jnp
from jax import lax
from jax.experimental import pallas as pl
from jax.experimental.pallas import tpu as pltpu
from jax.experimental.pallas import tpu_sc as plsc

DIM_IN = 1024
DIM_HIDDEN = 512
NUM_SUBKEYS = 256
TOP_K = 16
NUM_HEADS = 4
SUBKEY = DIM_HIDDEN // 2
CTX = 2048
EPS = 1e-5
NFEAT = DIM_HIDDEN * NUM_HEADS

TOK_BLK = 256
NUM_BLKS = CTX // TOK_BLK

NEG = -3.0e38


def _qstats_kernel(x_ref, wq_ref, q_ref, stats_ref):
    step = pl.program_id(0)
    qb = lax.dot_general(x_ref[...], wq_ref[...], (((1,), (1,)), ((), ())),
                         preferred_element_type=jnp.float32)
    q_ref[...] = qb

    @pl.when(step == 0)
    def _():
        stats_ref[...] = jnp.zeros_like(stats_ref)

    s1 = jnp.sum(qb, axis=0, keepdims=True)
    s2 = jnp.sum(qb * qb, axis=0, keepdims=True)
    stats_ref[...] += jnp.concatenate([s1, s2], axis=0)


def _top16_t(scores, rowids):
    vals, idxs = [], []
    sc = scores
    for _ in range(TOP_K):
        m = jnp.max(sc, axis=0, keepdims=True)
        am = jnp.min(jnp.where(sc >= m, rowids, jnp.int32(1 << 30)),
                     axis=0, keepdims=True)
        vals.append(m)
        idxs.append(am)
        sc = jnp.where(rowids == am, NEG, sc)
    return jnp.concatenate(vals, axis=0), jnp.concatenate(idxs, axis=0)


def _select_kernel(q_ref, stats_ref, gamma_ref, beta_ref, kl_ref, kr_ref,
                   vidx_ref, wsp_ref):
    n = jnp.float32(CTX)
    mean = stats_ref[0:1, :] / n
    var = stats_ref[1:2, :] / n - mean * mean
    scale = gamma_ref[0:1, :] * lax.rsqrt(var + EPS)
    qn = (q_ref[...] - mean) * scale + beta_ref[0:1, :]

    rowk = lax.broadcasted_iota(jnp.int32, (NUM_SUBKEYS, TOK_BLK), 0)
    rowp = lax.broadcasted_iota(jnp.int32, (TOP_K * TOP_K, TOK_BLK), 0)
    prow = lax.broadcasted_iota(jnp.int32, (TOP_K, TOP_K * TOP_K), 1)
    krow = lax.broadcasted_iota(jnp.int32, (TOP_K, TOP_K * TOP_K), 0)
    el = (prow // TOP_K == krow).astype(jnp.float32)
    er = (prow % TOP_K == krow).astype(jnp.float32)
    icol = lax.broadcasted_iota(jnp.int32, (TOP_K, TOK_BLK), 0)

    vidx_parts, wagg_parts = [], []
    for h in range(NUM_HEADS):
        ql = qn[:, h * DIM_HIDDEN:h * DIM_HIDDEN + SUBKEY]
        qr = qn[:, h * DIM_HIDDEN + SUBKEY:(h + 1) * DIM_HIDDEN]
        scl = lax.dot_general(kl_ref[h], ql, (((1,), (1,)), ((), ())),
                              preferred_element_type=jnp.float32)
        scr = lax.dot_general(kr_ref[h], qr, (((1,), (1,)), ((), ())),
                              preferred_element_type=jnp.float32)
        vl, il = _top16_t(scl, rowk)
        vr, ir = _top16_t(scr, rowk)

        ps = (lax.dot_general(el, vl, (((0,), (0,)), ((), ())),
                              preferred_element_type=jnp.float32,
                              precision=lax.Precision.HIGHEST) +
              lax.dot_general(er, vr, (((0,), (0,)), ((), ())),
                              preferred_element_type=jnp.float32,
                              precision=lax.Precision.HIGHEST))
        pv, pp = _top16_t(ps, rowp)

        pm = jnp.max(pv, axis=0, keepdims=True)
        e = jnp.exp(pv - pm)
        w = e / jnp.sum(e, axis=0, keepdims=True)

        pi = pp // TOP_K
        wagg = jnp.zeros((TOP_K, TOK_BLK), jnp.float32)
        for k in range(TOP_K):
            wagg = wagg + jnp.where(pi[k:k + 1, :] == icol,
                                    w[k:k + 1, :], 0.0)

        vidx_parts.append((il * NUM_SUBKEYS + ir).astype(jnp.float32))
        wagg_parts.append(wagg)

    vidx_t = jnp.concatenate(vidx_parts, axis=0)
    wagg_t = jnp.concatenate(wagg_parts, axis=0)
    ident = (lax.broadcasted_iota(jnp.int32, (TOK_BLK, TOK_BLK), 0) ==
             lax.broadcasted_iota(jnp.int32, (TOK_BLK, TOK_BLK), 1)
             ).astype(jnp.float32)
    vidx_f = lax.dot_general(ident, vidx_t, (((1,), (1,)), ((), ())),
                             preferred_element_type=jnp.float32,
                             precision=lax.Precision.HIGHEST)
    wagg2 = lax.dot_general(ident, wagg_t, (((1,), (1,)), ((), ())),
                            preferred_element_type=jnp.float32,
                            precision=lax.Precision.HIGHEST)
    vidx_ref[...] = vidx_f.astype(jnp.int32)
    wsp_ref[...] = jnp.repeat(wagg2, 16, axis=1)


TOK_PER_W = 64


def _sc_gather_body(vt_hbm, idx_hbm, w_hbm, out_hbm,
                    idx_v, buf0, buf1, wv0, wv1, acc_v,
                    g0, g1, ws0, ws1):
    nc = 2
    wid = lax.axis_index("s") * nc + lax.axis_index("c")
    base = wid * TOK_PER_W
    pltpu.sync_copy(idx_hbm.at[pl.ds(base, TOK_PER_W)], idx_v)
    zero = jnp.zeros((16,), jnp.float32)

    def fire_half0(t):
        pltpu.async_copy(vt_hbm.at[idx_v.at[t, pl.ds(0, 32)]], buf0, g0)

    def fire_half1(t):
        pltpu.async_copy(vt_hbm.at[idx_v.at[t, pl.ds(32, 32)]], buf1, g1)

    def accum(buf, w_v, roff, first):
        for cg in range(4):
            def row_body(r, accs):
                wv = w_v[roff + r]
                return tuple(
                    accs[g] + wv * buf[r, pl.ds((cg * 16 + g) * 16, 16)]
                    for g in range(16))
            accs = lax.fori_loop(0, 32, row_body, (zero,) * 16)
            for g in range(16):
                sl = pl.ds((cg * 16 + g) * 16, 16)
                if first:
                    acc_v[sl] = accs[g]
                else:
                    acc_v[sl] += accs[g]

    def one_token(t, w_v, wsem, wsem_next, wv_next):
        pltpu.make_async_copy(vt_hbm.at[idx_v.at[t, pl.ds(0, 32)]],
                              buf0, g0).wait()
        pltpu.make_async_copy(w_hbm.at[base + t], w_v, wsem).wait()
        fire_half1(t)
        accum(buf0, w_v, 0, True)

        @pl.when(t + 1 < TOK_PER_W)
        def _():
            fire_half0(t + 1)
            pltpu.async_copy(w_hbm.at[base + t + 1], wv_next, wsem_next)

        pltpu.make_async_copy(vt_hbm.at[idx_v.at[t, pl.ds(32, 32)]],
                              buf1, g1).wait()
        accum(buf1, w_v, 32, False)
        pltpu.sync_copy(acc_v, out_hbm.at[base + t])

    fire_half0(0)
    pltpu.async_copy(w_hbm.at[base], wv0, ws0)

    def pair_body(u, carry):
        one_token(2 * u, wv0, ws0, ws1, wv1)
        one_token(2 * u + 1, wv1, ws1, ws0, wv0)
        return carry

    lax.fori_loop(0, TOK_PER_W // 2, pair_body, 0)


def _sc_gather(value_table, vidx, wsp):
    mesh = plsc.VectorSubcoreMesh(core_axis_name="c", subcore_axis_name="s")
    kern = functools.partial(
        pl.kernel,
        mesh=mesh,
        out_type=jax.ShapeDtypeStruct((CTX, DIM_IN), jnp.float32),
        scratch_types=[
            pltpu.VMEM((TOK_PER_W, 64), jnp.int32),
            pltpu.VMEM((32, DIM_IN), jnp.float32),
            pltpu.VMEM((32, DIM_IN), jnp.float32),
            pltpu.VMEM((64, 16), jnp.float32),
            pltpu.VMEM((64, 16), jnp.float32),
            pltpu.VMEM((DIM_IN,), jnp.float32),
            pltpu.SemaphoreType.DMA,
            pltpu.SemaphoreType.DMA,
            pltpu.SemaphoreType.DMA,
            pltpu.SemaphoreType.DMA,
        ],
    )(_sc_gather_body)
    return kern(value_table, vidx, wsp)


def kernel(x, Wq, gamma, beta, keyl, keyr, value_table):
    b, c, _ = x.shape
    x2 = x.reshape(CTX, DIM_IN)

    q, stats = pl.pallas_call(
        _qstats_kernel,
        grid=(NUM_BLKS,),
        in_specs=[
            pl.BlockSpec((TOK_BLK, DIM_IN), lambda i: (i, 0)),
            pl.BlockSpec((NFEAT, DIM_IN), lambda i: (0, 0)),
        ],
        out_specs=[
            pl.BlockSpec((TOK_BLK, NFEAT), lambda i: (i, 0)),
            pl.BlockSpec((2, NFEAT), lambda i: (0, 0)),
        ],
        out_shape=[
            jax.ShapeDtypeStruct((CTX, NFEAT), jnp.float32),
            jax.ShapeDtypeStruct((2, NFEAT), jnp.float32),
        ],
    )(x2, Wq)

    vidx, wsp = pl.pallas_call(
        _select_kernel,
        grid=(NUM_BLKS,),
        in_specs=[
            pl.BlockSpec((TOK_BLK, NFEAT), lambda i: (i, 0)),
            pl.BlockSpec((2, NFEAT), lambda i: (0, 0)),
            pl.BlockSpec((1, NFEAT), lambda i: (0, 0)),
            pl.BlockSpec((1, NFEAT), lambda i: (0, 0)),
            pl.BlockSpec((NUM_HEADS, NUM_SUBKEYS, SUBKEY), lambda i: (0, 0, 0)),
            pl.BlockSpec((NUM_HEADS, NUM_SUBKEYS, SUBKEY), lambda i: (0, 0, 0)),
        ],
        out_specs=[
            pl.BlockSpec((TOK_BLK, NUM_HEADS * TOP_K), lambda i: (i, 0)),
            pl.BlockSpec((TOK_BLK, NUM_HEADS * TOP_K * 16), lambda i: (i, 0)),
        ],
        out_shape=[
            jax.ShapeDtypeStruct((CTX, NUM_HEADS * TOP_K), jnp.int32),
            jax.ShapeDtypeStruct((CTX, NUM_HEADS * TOP_K * 16), jnp.float32),
        ],
    )(q, stats, gamma.reshape(1, NFEAT), beta.reshape(1, NFEAT), keyl, keyr)

    wsp3 = wsp.reshape(CTX, NUM_HEADS * TOP_K, 16)
    out = _sc_gather(value_table, vidx, wsp3)
    return out.reshape(b, c, DIM_IN)


# speedup vs baseline: 8.9506x; 1.1383x over previous
import functools

import jax
import jax.numpy as jnp
from jax import lax
from jax.experimental import pallas as pl
from jax.experimental.pallas import tpu as pltpu
from jax.experimental.pallas import tpu_sc as plsc

DIM_IN = 1024
DIM_HIDDEN = 512
NUM_SUBKEYS = 256
TOP_K = 16
NUM_HEADS = 4
SUBKEY = DIM_HIDDEN // 2
CTX = 2048
EPS = 1e-5
NFEAT = DIM_HIDDEN * NUM_HEADS

TOK_BLK = 256
NUM_BLKS = CTX // TOK_BLK

NEG = -3.0e38


def _qstats_kernel(x_ref, wq_ref, q_ref, stats_ref):
    step = pl.program_id(0)
    qb = lax.dot_general(x_ref[...], wq_ref[...], (((1,), (1,)), ((), ())),
                         preferred_element_type=jnp.float32)
    q_ref[...] = qb

    @pl.when(step == 0)
    def _():
        stats_ref[...] = jnp.zeros_like(stats_ref)

    s1 = jnp.sum(qb, axis=0, keepdims=True)
    s2 = jnp.sum(qb * qb, axis=0, keepdims=True)
    stats_ref[...] += jnp.concatenate([s1, s2], axis=0)


def _top16_t(scores, rowids):
    vals, idxs = [], []
    sc = scores
    for _ in range(TOP_K):
        m = jnp.max(sc, axis=0, keepdims=True)
        am = jnp.min(jnp.where(sc >= m, rowids, jnp.int32(1 << 30)),
                     axis=0, keepdims=True)
        vals.append(m)
        idxs.append(am)
        sc = jnp.where(rowids == am, NEG, sc)
    return jnp.concatenate(vals, axis=0), jnp.concatenate(idxs, axis=0)


def _select_kernel(q_ref, stats_ref, gamma_ref, beta_ref, kl_ref, kr_ref,
                   vidx_ref, wsp_ref):
    n = jnp.float32(CTX)
    mean = stats_ref[0:1, :] / n
    var = stats_ref[1:2, :] / n - mean * mean
    scale = gamma_ref[0:1, :] * lax.rsqrt(var + EPS)
    qn = (q_ref[...] - mean) * scale + beta_ref[0:1, :]

    rowk = lax.broadcasted_iota(jnp.int32, (NUM_SUBKEYS, TOK_BLK), 0)
    rowp = lax.broadcasted_iota(jnp.int32, (TOP_K * TOP_K, TOK_BLK), 0)
    prow = lax.broadcasted_iota(jnp.int32, (TOP_K, TOP_K * TOP_K), 1)
    krow = lax.broadcasted_iota(jnp.int32, (TOP_K, TOP_K * TOP_K), 0)
    el = (prow // TOP_K == krow).astype(jnp.float32)
    er = (prow % TOP_K == krow).astype(jnp.float32)
    icol = lax.broadcasted_iota(jnp.int32, (TOP_K, TOK_BLK), 0)

    vidx_parts, wagg_parts = [], []
    for h in range(NUM_HEADS):
        ql = qn[:, h * DIM_HIDDEN:h * DIM_HIDDEN + SUBKEY]
        qr = qn[:, h * DIM_HIDDEN + SUBKEY:(h + 1) * DIM_HIDDEN]
        scl = lax.dot_general(kl_ref[h], ql, (((1,), (1,)), ((), ())),
                              preferred_element_type=jnp.float32)
        scr = lax.dot_general(kr_ref[h], qr, (((1,), (1,)), ((), ())),
                              preferred_element_type=jnp.float32)
        vl, il = _top16_t(scl, rowk)
        vr, ir = _top16_t(scr, rowk)

        ps = (lax.dot_general(el, vl, (((0,), (0,)), ((), ())),
                              preferred_element_type=jnp.float32,
                              precision=lax.Precision.HIGHEST) +
              lax.dot_general(er, vr, (((0,), (0,)), ((), ())),
                              preferred_element_type=jnp.float32,
                              precision=lax.Precision.HIGHEST))
        pv, pp = _top16_t(ps, rowp)

        pm = jnp.max(pv, axis=0, keepdims=True)
        e = jnp.exp(pv - pm)
        w = e / jnp.sum(e, axis=0, keepdims=True)

        pi = pp // TOP_K
        wagg = jnp.zeros((TOP_K, TOK_BLK), jnp.float32)
        for k in range(TOP_K):
            wagg = wagg + jnp.where(pi[k:k + 1, :] == icol,
                                    w[k:k + 1, :], 0.0)

        vidx_parts.append((il * NUM_SUBKEYS + ir).astype(jnp.float32))
        wagg_parts.append(wagg)

    vidx_t = jnp.concatenate(vidx_parts, axis=0)
    wagg_t = jnp.concatenate(wagg_parts, axis=0)
    ident = (lax.broadcasted_iota(jnp.int32, (TOK_BLK, TOK_BLK), 0) ==
             lax.broadcasted_iota(jnp.int32, (TOK_BLK, TOK_BLK), 1)
             ).astype(jnp.float32)
    vidx_f = lax.dot_general(ident, vidx_t, (((1,), (1,)), ((), ())),
                             preferred_element_type=jnp.float32,
                             precision=lax.Precision.HIGHEST)
    wagg2 = lax.dot_general(ident, wagg_t, (((1,), (1,)), ((), ())),
                            preferred_element_type=jnp.float32,
                            precision=lax.Precision.HIGHEST)
    vidx_ref[...] = vidx_f.astype(jnp.int32)
    wsp_ref[...] = jnp.repeat(wagg2, 16, axis=1)


def _sc_gather_body(vt_hbm, idx_hbm, w_hbm, out_hbm,
                    idx_v, buf0, buf1, wv0, wv1, acc_v,
                    g0, g1, ws0, ws1, *, tok_per_w):
    TOK_PER_W = tok_per_w
    nc = 2
    wid = lax.axis_index("s") * nc + lax.axis_index("c")
    base = wid * TOK_PER_W
    pltpu.sync_copy(idx_hbm.at[pl.ds(base, TOK_PER_W)], idx_v)
    zero = jnp.zeros((16,), jnp.float32)

    def fire_half0(t):
        pltpu.async_copy(vt_hbm.at[idx_v.at[t, pl.ds(0, 32)]], buf0, g0)

    def fire_half1(t):
        pltpu.async_copy(vt_hbm.at[idx_v.at[t, pl.ds(32, 32)]], buf1, g1)

    def accum(buf, w_v, roff, first):
        for cg in range(4):
            def row_body(r, accs):
                wv = w_v[roff + r]
                return tuple(
                    accs[g] + wv * buf[r, pl.ds((cg * 16 + g) * 16, 16)]
                    for g in range(16))
            accs = lax.fori_loop(0, 32, row_body, (zero,) * 16)
            for g in range(16):
                sl = pl.ds((cg * 16 + g) * 16, 16)
                if first:
                    acc_v[sl] = accs[g]
                else:
                    acc_v[sl] += accs[g]

    def one_token(t, w_v, wsem, wsem_next, wv_next):
        pltpu.make_async_copy(vt_hbm.at[idx_v.at[t, pl.ds(0, 32)]],
                              buf0, g0).wait()
        pltpu.make_async_copy(w_hbm.at[base + t], w_v, wsem).wait()
        fire_half1(t)
        accum(buf0, w_v, 0, True)

        @pl.when(t + 1 < TOK_PER_W)
        def _():
            fire_half0(t + 1)
            pltpu.async_copy(w_hbm.at[base + t + 1], wv_next, wsem_next)

        pltpu.make_async_copy(vt_hbm.at[idx_v.at[t, pl.ds(32, 32)]],
                              buf1, g1).wait()
        accum(buf1, w_v, 32, False)
        pltpu.sync_copy(acc_v, out_hbm.at[base + t])

    fire_half0(0)
    pltpu.async_copy(w_hbm.at[base], wv0, ws0)

    def pair_body(u, carry):
        one_token(2 * u, wv0, ws0, ws1, wv1)
        one_token(2 * u + 1, wv1, ws1, ws0, wv0)
        return carry

    lax.fori_loop(0, TOK_PER_W // 2, pair_body, 0)


def _sc_gather(value_table, vidx, wsp):
    tokens = vidx.shape[0]
    tok_per_w = tokens // 32
    mesh = plsc.VectorSubcoreMesh(core_axis_name="c", subcore_axis_name="s")
    kern = functools.partial(
        pl.kernel,
        mesh=mesh,
        out_type=jax.ShapeDtypeStruct((tokens, DIM_IN), jnp.float32),
        scratch_types=[
            pltpu.VMEM((tok_per_w, 64), jnp.int32),
            pltpu.VMEM((32, DIM_IN), jnp.float32),
            pltpu.VMEM((32, DIM_IN), jnp.float32),
            pltpu.VMEM((64, 16), jnp.float32),
            pltpu.VMEM((64, 16), jnp.float32),
            pltpu.VMEM((DIM_IN,), jnp.float32),
            pltpu.SemaphoreType.DMA,
            pltpu.SemaphoreType.DMA,
            pltpu.SemaphoreType.DMA,
            pltpu.SemaphoreType.DMA,
        ],
    )(functools.partial(_sc_gather_body, tok_per_w=tok_per_w))
    return kern(value_table, vidx, wsp)


def kernel(x, Wq, gamma, beta, keyl, keyr, value_table):
    b, c, _ = x.shape
    x2 = x.reshape(CTX, DIM_IN)

    q, stats = pl.pallas_call(
        _qstats_kernel,
        grid=(NUM_BLKS,),
        in_specs=[
            pl.BlockSpec((TOK_BLK, DIM_IN), lambda i: (i, 0)),
            pl.BlockSpec((NFEAT, DIM_IN), lambda i: (0, 0)),
        ],
        out_specs=[
            pl.BlockSpec((TOK_BLK, NFEAT), lambda i: (i, 0)),
            pl.BlockSpec((2, NFEAT), lambda i: (0, 0)),
        ],
        out_shape=[
            jax.ShapeDtypeStruct((CTX, NFEAT), jnp.float32),
            jax.ShapeDtypeStruct((2, NFEAT), jnp.float32),
        ],
    )(x2, Wq)

    nchunk = 2
    chunk = CTX // nchunk
    outs = []
    for ci in range(nchunk):
        qc = lax.slice(q, (ci * chunk, 0), ((ci + 1) * chunk, NFEAT))
        vidx, wsp = pl.pallas_call(
            _select_kernel,
            grid=(chunk // TOK_BLK,),
            in_specs=[
                pl.BlockSpec((TOK_BLK, NFEAT), lambda i: (i, 0)),
                pl.BlockSpec((2, NFEAT), lambda i: (0, 0)),
                pl.BlockSpec((1, NFEAT), lambda i: (0, 0)),
                pl.BlockSpec((1, NFEAT), lambda i: (0, 0)),
                pl.BlockSpec((NUM_HEADS, NUM_SUBKEYS, SUBKEY),
                             lambda i: (0, 0, 0)),
                pl.BlockSpec((NUM_HEADS, NUM_SUBKEYS, SUBKEY),
                             lambda i: (0, 0, 0)),
            ],
            out_specs=[
                pl.BlockSpec((TOK_BLK, NUM_HEADS * TOP_K), lambda i: (i, 0)),
                pl.BlockSpec((TOK_BLK, NUM_HEADS * TOP_K * 16),
                             lambda i: (i, 0)),
            ],
            out_shape=[
                jax.ShapeDtypeStruct((chunk, NUM_HEADS * TOP_K), jnp.int32),
                jax.ShapeDtypeStruct((chunk, NUM_HEADS * TOP_K * 16),
                                     jnp.float32),
            ],
        )(qc, stats, gamma.reshape(1, NFEAT), beta.reshape(1, NFEAT),
          keyl, keyr)
        wsp3 = wsp.reshape(chunk, NUM_HEADS * TOP_K, 16)
        outs.append(_sc_gather(value_table, vidx, wsp3))

    out = jnp.concatenate(outs, axis=0)
    return out.reshape(b, c, DIM_IN)


# speedup vs baseline: 9.4702x; 1.0581x over previous
import functools

import jax
import jax.numpy as jnp
from jax import lax
from jax.experimental import pallas as pl
from jax.experimental.pallas import tpu as pltpu
from jax.experimental.pallas import tpu_sc as plsc

DIM_IN = 1024
DIM_HIDDEN = 512
NUM_SUBKEYS = 256
TOP_K = 16
NUM_HEADS = 4
SUBKEY = DIM_HIDDEN // 2
CTX = 2048
EPS = 1e-5
NFEAT = DIM_HIDDEN * NUM_HEADS

TOK_BLK = 256
NUM_BLKS = CTX // TOK_BLK

NEG = -3.0e38


def _qstats_kernel(x_ref, wq_ref, q_ref, stats_ref):
    step = pl.program_id(0)
    qb = lax.dot_general(x_ref[...], wq_ref[...], (((1,), (1,)), ((), ())),
                         preferred_element_type=jnp.float32)
    q_ref[...] = qb

    @pl.when(step == 0)
    def _():
        stats_ref[...] = jnp.zeros_like(stats_ref)

    s1 = jnp.sum(qb, axis=0, keepdims=True)
    s2 = jnp.sum(qb * qb, axis=0, keepdims=True)
    stats_ref[...] += jnp.concatenate([s1, s2], axis=0)


def _top16_t(scores, rowids):
    vals, idxs = [], []
    sc = scores
    for _ in range(TOP_K):
        m = jnp.max(sc, axis=0, keepdims=True)
        am = jnp.min(jnp.where(sc >= m, rowids, jnp.int32(1 << 30)),
                     axis=0, keepdims=True)
        vals.append(m)
        idxs.append(am)
        sc = jnp.where(rowids == am, NEG, sc)
    return jnp.concatenate(vals, axis=0), jnp.concatenate(idxs, axis=0)


def _select_kernel(q_ref, stats_ref, gamma_ref, beta_ref, kl_ref, kr_ref,
                   vidx_ref, wsp_ref):
    n = jnp.float32(CTX)
    mean = stats_ref[0:1, :] / n
    var = stats_ref[1:2, :] / n - mean * mean
    scale = gamma_ref[0:1, :] * lax.rsqrt(var + EPS)
    qn = (q_ref[...] - mean) * scale + beta_ref[0:1, :]

    rowk = lax.broadcasted_iota(jnp.int32, (NUM_SUBKEYS, TOK_BLK), 0)
    rowp = lax.broadcasted_iota(jnp.int32, (TOP_K * TOP_K, TOK_BLK), 0)
    prow = lax.broadcasted_iota(jnp.int32, (TOP_K, TOP_K * TOP_K), 1)
    krow = lax.broadcasted_iota(jnp.int32, (TOP_K, TOP_K * TOP_K), 0)
    el = (prow // TOP_K == krow).astype(jnp.float32)
    er = (prow % TOP_K == krow).astype(jnp.float32)
    icol = lax.broadcasted_iota(jnp.int32, (TOP_K, TOK_BLK), 0)

    vidx_parts, wagg_parts = [], []
    for h in range(NUM_HEADS):
        ql = qn[:, h * DIM_HIDDEN:h * DIM_HIDDEN + SUBKEY]
        qr = qn[:, h * DIM_HIDDEN + SUBKEY:(h + 1) * DIM_HIDDEN]
        scl = lax.dot_general(kl_ref[h], ql, (((1,), (1,)), ((), ())),
                              preferred_element_type=jnp.float32)
        scr = lax.dot_general(kr_ref[h], qr, (((1,), (1,)), ((), ())),
                              preferred_element_type=jnp.float32)
        vl, il = _top16_t(scl, rowk)
        vr, ir = _top16_t(scr, rowk)

        ps = (lax.dot_general(el, vl, (((0,), (0,)), ((), ())),
                              preferred_element_type=jnp.float32,
                              precision=lax.Precision.HIGHEST) +
              lax.dot_general(er, vr, (((0,), (0,)), ((), ())),
                              preferred_element_type=jnp.float32,
                              precision=lax.Precision.HIGHEST))
        pv, pp = _top16_t(ps, rowp)

        pm = jnp.max(pv, axis=0, keepdims=True)
        e = jnp.exp(pv - pm)
        w = e / jnp.sum(e, axis=0, keepdims=True)

        pi = pp // TOP_K
        wagg = jnp.zeros((TOP_K, TOK_BLK), jnp.float32)
        for k in range(TOP_K):
            wagg = wagg + jnp.where(pi[k:k + 1, :] == icol,
                                    w[k:k + 1, :], 0.0)

        vidx_parts.append((il * NUM_SUBKEYS + ir).astype(jnp.float32))
        wagg_parts.append(wagg)

    vidx_t = jnp.concatenate(vidx_parts, axis=0)
    wagg_t = jnp.concatenate(wagg_parts, axis=0)
    ident = (lax.broadcasted_iota(jnp.int32, (TOK_BLK, TOK_BLK), 0) ==
             lax.broadcasted_iota(jnp.int32, (TOK_BLK, TOK_BLK), 1)
             ).astype(jnp.float32)
    vidx_f = lax.dot_general(ident, vidx_t, (((1,), (1,)), ((), ())),
                             preferred_element_type=jnp.float32,
                             precision=lax.Precision.HIGHEST)
    wagg2 = lax.dot_general(ident, wagg_t, (((1,), (1,)), ((), ())),
                            preferred_element_type=jnp.float32,
                            precision=lax.Precision.HIGHEST)
    vidx_ref[...] = vidx_f.astype(jnp.int32)
    wsp_ref[...] = jnp.repeat(wagg2, 16, axis=1)


def _sc_gather_body(vt_hbm, idx_hbm, w_hbm, out_hbm,
                    idx_v, buf0, buf1, wv0, wv1, acc_v,
                    g0, g1, ws0, ws1, *, tok_per_w):
    TOK_PER_W = tok_per_w
    nc = 2
    wid = lax.axis_index("s") * nc + lax.axis_index("c")
    base = wid * TOK_PER_W
    pltpu.sync_copy(idx_hbm.at[pl.ds(base, TOK_PER_W)], idx_v)
    zero = jnp.zeros((16,), jnp.float32)

    def fire_half0(t):
        pltpu.async_copy(vt_hbm.at[idx_v.at[t, pl.ds(0, 32)]], buf0, g0)

    def fire_half1(t):
        pltpu.async_copy(vt_hbm.at[idx_v.at[t, pl.ds(32, 32)]], buf1, g1)

    def accum(buf, w_v, roff, first):
        for cg in range(4):
            def row_body(r, accs):
                wv = w_v[roff + r]
                return tuple(
                    accs[g] + wv * buf[r, pl.ds((cg * 16 + g) * 16, 16)]
                    for g in range(16))
            accs = lax.fori_loop(0, 32, row_body, (zero,) * 16)
            for g in range(16):
                sl = pl.ds((cg * 16 + g) * 16, 16)
                if first:
                    acc_v[sl] = accs[g]
                else:
                    acc_v[sl] += accs[g]

    def one_token(t, w_v, wsem, wsem_next, wv_next):
        pltpu.make_async_copy(vt_hbm.at[idx_v.at[t, pl.ds(0, 32)]],
                              buf0, g0).wait()
        pltpu.make_async_copy(w_hbm.at[base + t], w_v, wsem).wait()
        fire_half1(t)
        accum(buf0, w_v, 0, True)

        @pl.when(t + 1 < TOK_PER_W)
        def _():
            fire_half0(t + 1)
            pltpu.async_copy(w_hbm.at[base + t + 1], wv_next, wsem_next)

        pltpu.make_async_copy(vt_hbm.at[idx_v.at[t, pl.ds(32, 32)]],
                              buf1, g1).wait()
        accum(buf1, w_v, 32, False)
        pltpu.sync_copy(acc_v, out_hbm.at[base + t])

    fire_half0(0)
    pltpu.async_copy(w_hbm.at[base], wv0, ws0)

    def pair_body(u, carry):
        one_token(2 * u, wv0, ws0, ws1, wv1)
        one_token(2 * u + 1, wv1, ws1, ws0, wv0)
        return carry

    lax.fori_loop(0, TOK_PER_W // 2, pair_body, 0)


def _sc_gather(value_table, vidx, wsp):
    tokens = vidx.shape[0]
    tok_per_w = tokens // 32
    mesh = plsc.VectorSubcoreMesh(core_axis_name="c", subcore_axis_name="s")
    kern = functools.partial(
        pl.kernel,
        mesh=mesh,
        out_type=jax.ShapeDtypeStruct((tokens, DIM_IN), jnp.float32),
        scratch_types=[
            pltpu.VMEM((tok_per_w, 64), jnp.int32),
            pltpu.VMEM((32, DIM_IN), jnp.float32),
            pltpu.VMEM((32, DIM_IN), jnp.float32),
            pltpu.VMEM((64, 16), jnp.float32),
            pltpu.VMEM((64, 16), jnp.float32),
            pltpu.VMEM((DIM_IN,), jnp.float32),
            pltpu.SemaphoreType.DMA,
            pltpu.SemaphoreType.DMA,
            pltpu.SemaphoreType.DMA,
            pltpu.SemaphoreType.DMA,
        ],
    )(functools.partial(_sc_gather_body, tok_per_w=tok_per_w))
    return kern(value_table, vidx, wsp)


def kernel(x, Wq, gamma, beta, keyl, keyr, value_table):
    b, c, _ = x.shape
    x2 = x.reshape(CTX, DIM_IN)

    q, stats = pl.pallas_call(
        _qstats_kernel,
        grid=(NUM_BLKS,),
        in_specs=[
            pl.BlockSpec((TOK_BLK, DIM_IN), lambda i: (i, 0)),
            pl.BlockSpec((NFEAT, DIM_IN), lambda i: (0, 0)),
        ],
        out_specs=[
            pl.BlockSpec((TOK_BLK, NFEAT), lambda i: (i, 0)),
            pl.BlockSpec((2, NFEAT), lambda i: (0, 0)),
        ],
        out_shape=[
            jax.ShapeDtypeStruct((CTX, NFEAT), jnp.float32),
            jax.ShapeDtypeStruct((2, NFEAT), jnp.float32),
        ],
    )(x2, Wq)

    nchunk = 4
    chunk = CTX // nchunk
    outs = []
    for ci in range(nchunk):
        qc = lax.slice(q, (ci * chunk, 0), ((ci + 1) * chunk, NFEAT))
        vidx, wsp = pl.pallas_call(
            _select_kernel,
            grid=(chunk // TOK_BLK,),
            in_specs=[
                pl.BlockSpec((TOK_BLK, NFEAT), lambda i: (i, 0)),
                pl.BlockSpec((2, NFEAT), lambda i: (0, 0)),
                pl.BlockSpec((1, NFEAT), lambda i: (0, 0)),
                pl.BlockSpec((1, NFEAT), lambda i: (0, 0)),
                pl.BlockSpec((NUM_HEADS, NUM_SUBKEYS, SUBKEY),
                             lambda i: (0, 0, 0)),
                pl.BlockSpec((NUM_HEADS, NUM_SUBKEYS, SUBKEY),
                             lambda i: (0, 0, 0)),
            ],
            out_specs=[
                pl.BlockSpec((TOK_BLK, NUM_HEADS * TOP_K), lambda i: (i, 0)),
                pl.BlockSpec((TOK_BLK, NUM_HEADS * TOP_K * 16),
                             lambda i: (i, 0)),
            ],
            out_shape=[
                jax.ShapeDtypeStruct((chunk, NUM_HEADS * TOP_K), jnp.int32),
                jax.ShapeDtypeStruct((chunk, NUM_HEADS * TOP_K * 16),
                                     jnp.float32),
            ],
        )(qc, stats, gamma.reshape(1, NFEAT), beta.reshape(1, NFEAT),
          keyl, keyr)
        wsp3 = wsp.reshape(chunk, NUM_HEADS * TOP_K, 16)
        outs.append(_sc_gather(value_table, vidx, wsp3))

    out = jnp.concatenate(outs, axis=0)
    return out.reshape(b, c, DIM_IN)


# speedup vs baseline: 9.4745x; 1.0004x over previous
import functools

import jax
import jax.numpy as jnp
from jax import lax
from jax.experimental import pallas as pl
from jax.experimental.pallas import tpu as pltpu
from jax.experimental.pallas import tpu_sc as plsc

DIM_IN = 1024
DIM_HIDDEN = 512
NUM_SUBKEYS = 256
TOP_K = 16
NUM_HEADS = 4
SUBKEY = DIM_HIDDEN // 2
CTX = 2048
EPS = 1e-5
NFEAT = DIM_HIDDEN * NUM_HEADS

TOK_BLK = 256
NUM_BLKS = CTX // TOK_BLK

NEG = -3.0e38


def _qstats_kernel(x_ref, wq_ref, q_ref, stats_ref):
    step = pl.program_id(0)
    qb = lax.dot_general(x_ref[...], wq_ref[...], (((1,), (1,)), ((), ())),
                         preferred_element_type=jnp.float32)
    q_ref[...] = qb

    @pl.when(step == 0)
    def _():
        stats_ref[...] = jnp.zeros_like(stats_ref)

    s1 = jnp.sum(qb, axis=0, keepdims=True)
    s2 = jnp.sum(qb * qb, axis=0, keepdims=True)
    stats_ref[...] += jnp.concatenate([s1, s2], axis=0)


def _top16_t(scores, rowids):
    vals, idxs = [], []
    sc = scores
    for _ in range(TOP_K):
        m = jnp.max(sc, axis=0, keepdims=True)
        am = jnp.min(jnp.where(sc >= m, rowids, jnp.int32(1 << 30)),
                     axis=0, keepdims=True)
        vals.append(m)
        idxs.append(am)
        sc = jnp.where(rowids == am, NEG, sc)
    return jnp.concatenate(vals, axis=0), jnp.concatenate(idxs, axis=0)


def _select_kernel(q_ref, stats_ref, gamma_ref, beta_ref, kl_ref, kr_ref,
                   vidx_ref, wsp_ref):
    n = jnp.float32(CTX)
    mean = stats_ref[0:1, :] / n
    var = stats_ref[1:2, :] / n - mean * mean
    qn = ((q_ref[...] - mean) / jnp.sqrt(var + EPS) * gamma_ref[0:1, :]
          + beta_ref[0:1, :])

    rowk = lax.broadcasted_iota(jnp.int32, (NUM_SUBKEYS, TOK_BLK), 0)
    rowp = lax.broadcasted_iota(jnp.int32, (TOP_K * TOP_K, TOK_BLK), 0)
    prow = lax.broadcasted_iota(jnp.int32, (TOP_K, TOP_K * TOP_K), 1)
    krow = lax.broadcasted_iota(jnp.int32, (TOP_K, TOP_K * TOP_K), 0)
    el = (prow // TOP_K == krow).astype(jnp.float32)
    er = (prow % TOP_K == krow).astype(jnp.float32)
    icol = lax.broadcasted_iota(jnp.int32, (TOP_K, TOK_BLK), 0)

    vidx_parts, wagg_parts = [], []
    for h in range(NUM_HEADS):
        ql = qn[:, h * DIM_HIDDEN:h * DIM_HIDDEN + SUBKEY]
        qr = qn[:, h * DIM_HIDDEN + SUBKEY:(h + 1) * DIM_HIDDEN]
        scl = lax.dot_general(kl_ref[h], ql, (((1,), (1,)), ((), ())),
                              preferred_element_type=jnp.float32)
        scr = lax.dot_general(kr_ref[h], qr, (((1,), (1,)), ((), ())),
                              preferred_element_type=jnp.float32)
        vl, il = _top16_t(scl, rowk)
        vr, ir = _top16_t(scr, rowk)

        ps = (lax.dot_general(el, vl, (((0,), (0,)), ((), ())),
                              preferred_element_type=jnp.float32,
                              precision=lax.Precision.HIGHEST) +
              lax.dot_general(er, vr, (((0,), (0,)), ((), ())),
                              preferred_element_type=jnp.float32,
                              precision=lax.Precision.HIGHEST))
        pv, pp = _top16_t(ps, rowp)

        pm = jnp.max(pv, axis=0, keepdims=True)
        e = jnp.exp(pv - pm)
        w = e / jnp.sum(e, axis=0, keepdims=True)

        pi = pp // TOP_K
        wagg = jnp.zeros((TOP_K, TOK_BLK), jnp.float32)
        for k in range(TOP_K):
            wagg = wagg + jnp.where(pi[k:k + 1, :] == icol,
                                    w[k:k + 1, :], 0.0)

        vidx_parts.append((il * NUM_SUBKEYS + ir).astype(jnp.float32))
        wagg_parts.append(wagg)

    vidx_t = jnp.concatenate(vidx_parts, axis=0)
    wagg_t = jnp.concatenate(wagg_parts, axis=0)
    ident = (lax.broadcasted_iota(jnp.int32, (TOK_BLK, TOK_BLK), 0) ==
             lax.broadcasted_iota(jnp.int32, (TOK_BLK, TOK_BLK), 1)
             ).astype(jnp.float32)
    vidx_f = lax.dot_general(ident, vidx_t, (((1,), (1,)), ((), ())),
                             preferred_element_type=jnp.float32,
                             precision=lax.Precision.HIGHEST)
    wagg2 = lax.dot_general(ident, wagg_t, (((1,), (1,)), ((), ())),
                            preferred_element_type=jnp.float32,
                            precision=lax.Precision.HIGHEST)
    vidx_ref[...] = vidx_f.astype(jnp.int32)
    wsp_ref[...] = jnp.repeat(wagg2, 16, axis=1)


def _sc_gather_body(vt_hbm, idx_hbm, w_hbm, out_hbm,
                    idx_v, buf0, buf1, wv0, wv1, acc_v,
                    g0, g1, ws0, ws1, *, tok_per_w):
    TOK_PER_W = tok_per_w
    nc = 2
    wid = lax.axis_index("s") * nc + lax.axis_index("c")
    base = wid * TOK_PER_W
    pltpu.sync_copy(idx_hbm.at[pl.ds(base, TOK_PER_W)], idx_v)
    zero = jnp.zeros((16,), jnp.float32)

    def fire_half0(t):
        pltpu.async_copy(vt_hbm.at[idx_v.at[t, pl.ds(0, 32)]], buf0, g0)

    def fire_half1(t):
        pltpu.async_copy(vt_hbm.at[idx_v.at[t, pl.ds(32, 32)]], buf1, g1)

    def accum(buf, w_v, roff, first):
        for cg in range(4):
            def row_body(r, accs):
                wv = w_v[roff + r]
                return tuple(
                    accs[g] + wv * buf[r, pl.ds((cg * 16 + g) * 16, 16)]
                    for g in range(16))
            accs = lax.fori_loop(0, 32, row_body, (zero,) * 16)
            for g in range(16):
                sl = pl.ds((cg * 16 + g) * 16, 16)
                if first:
                    acc_v[sl] = accs[g]
                else:
                    acc_v[sl] += accs[g]

    def one_token(t, w_v, wsem, wsem_next, wv_next):
        pltpu.make_async_copy(vt_hbm.at[idx_v.at[t, pl.ds(0, 32)]],
                              buf0, g0).wait()
        pltpu.make_async_copy(w_hbm.at[base + t], w_v, wsem).wait()
        fire_half1(t)
        accum(buf0, w_v, 0, True)

        @pl.when(t + 1 < TOK_PER_W)
        def _():
            fire_half0(t + 1)
            pltpu.async_copy(w_hbm.at[base + t + 1], wv_next, wsem_next)

        pltpu.make_async_copy(vt_hbm.at[idx_v.at[t, pl.ds(32, 32)]],
                              buf1, g1).wait()
        accum(buf1, w_v, 32, False)
        pltpu.sync_copy(acc_v, out_hbm.at[base + t])

    fire_half0(0)
    pltpu.async_copy(w_hbm.at[base], wv0, ws0)

    def pair_body(u, carry):
        one_token(2 * u, wv0, ws0, ws1, wv1)
        one_token(2 * u + 1, wv1, ws1, ws0, wv0)
        return carry

    lax.fori_loop(0, TOK_PER_W // 2, pair_body, 0)


def _sc_gather(value_table, vidx, wsp):
    tokens = vidx.shape[0]
    tok_per_w = tokens // 32
    mesh = plsc.VectorSubcoreMesh(core_axis_name="c", subcore_axis_name="s")
    kern = functools.partial(
        pl.kernel,
        mesh=mesh,
        out_type=jax.ShapeDtypeStruct((tokens, DIM_IN), jnp.float32),
        scratch_types=[
            pltpu.VMEM((tok_per_w, 64), jnp.int32),
            pltpu.VMEM((32, DIM_IN), jnp.float32),
            pltpu.VMEM((32, DIM_IN), jnp.float32),
            pltpu.VMEM((64, 16), jnp.float32),
            pltpu.VMEM((64, 16), jnp.float32),
            pltpu.VMEM((DIM_IN,), jnp.float32),
            pltpu.SemaphoreType.DMA,
            pltpu.SemaphoreType.DMA,
            pltpu.SemaphoreType.DMA,
            pltpu.SemaphoreType.DMA,
        ],
    )(functools.partial(_sc_gather_body, tok_per_w=tok_per_w))
    return kern(value_table, vidx, wsp)


def kernel(x, Wq, gamma, beta, keyl, keyr, value_table):
    b, c, _ = x.shape
    x2 = x.reshape(CTX, DIM_IN)

    q, stats = pl.pallas_call(
        _qstats_kernel,
        grid=(NUM_BLKS,),
        in_specs=[
            pl.BlockSpec((TOK_BLK, DIM_IN), lambda i: (i, 0)),
            pl.BlockSpec((NFEAT, DIM_IN), lambda i: (0, 0)),
        ],
        out_specs=[
            pl.BlockSpec((TOK_BLK, NFEAT), lambda i: (i, 0)),
            pl.BlockSpec((2, NFEAT), lambda i: (0, 0)),
        ],
        out_shape=[
            jax.ShapeDtypeStruct((CTX, NFEAT), jnp.float32),
            jax.ShapeDtypeStruct((2, NFEAT), jnp.float32),
        ],
    )(x2, Wq)

    nchunk = 4
    chunk = CTX // nchunk
    outs = []
    for ci in range(nchunk):
        qc = lax.slice(q, (ci * chunk, 0), ((ci + 1) * chunk, NFEAT))
        vidx, wsp = pl.pallas_call(
            _select_kernel,
            grid=(chunk // TOK_BLK,),
            in_specs=[
                pl.BlockSpec((TOK_BLK, NFEAT), lambda i: (i, 0)),
                pl.BlockSpec((2, NFEAT), lambda i: (0, 0)),
                pl.BlockSpec((1, NFEAT), lambda i: (0, 0)),
                pl.BlockSpec((1, NFEAT), lambda i: (0, 0)),
                pl.BlockSpec((NUM_HEADS, NUM_SUBKEYS, SUBKEY),
                             lambda i: (0, 0, 0)),
                pl.BlockSpec((NUM_HEADS, NUM_SUBKEYS, SUBKEY),
                             lambda i: (0, 0, 0)),
            ],
            out_specs=[
                pl.BlockSpec((TOK_BLK, NUM_HEADS * TOP_K), lambda i: (i, 0)),
                pl.BlockSpec((TOK_BLK, NUM_HEADS * TOP_K * 16),
                             lambda i: (i, 0)),
            ],
            out_shape=[
                jax.ShapeDtypeStruct((chunk, NUM_HEADS * TOP_K), jnp.int32),
                jax.ShapeDtypeStruct((chunk, NUM_HEADS * TOP_K * 16),
                                     jnp.float32),
            ],
        )(qc, stats, gamma.reshape(1, NFEAT), beta.reshape(1, NFEAT),
          keyl, keyr)
        wsp3 = wsp.reshape(chunk, NUM_HEADS * TOP_K, 16)
        outs.append(_sc_gather(value_table, vidx, wsp3))

    out = jnp.concatenate(outs, axis=0)
    return out.reshape(b, c, DIM_IN)
